# baseline (speedup 1.0000x reference)
import functools

import jax
import jax.numpy as jnp
from jax.experimental import pallas as pl
from jax.experimental.pallas import tpu as pltpu

B = 1024
S = 16
A = 100000
ABLK = 512
NBLK = (A + ABLK - 1) // ABLK


def _ac_body(state_ref, wp_ref, bp_ref, act_ref, wv_ref, bv_ref,
             value_ref, lp_ref, ent_ref,
             m_ref, s_ref, t_ref, la_ref):
    j = pl.program_id(0)

    @pl.when(j == 0)
    def _init():
        m_ref[...] = jnp.full((B, 1), -1e30, jnp.float32)
        s_ref[...] = jnp.zeros((B, 1), jnp.float32)
        t_ref[...] = jnp.zeros((B, 1), jnp.float32)
        la_ref[...] = jnp.zeros((B, 1), jnp.float32)
        value_ref[...] = (
            jnp.dot(state_ref[...], wv_ref[...],
                    preferred_element_type=jnp.float32)
            + bv_ref[0, 0]
        )

    l = (
        jnp.dot(state_ref[...], wp_ref[...],
                preferred_element_type=jnp.float32)
        + bp_ref[...]
    )

    cols = jax.lax.broadcasted_iota(jnp.int32, (B, ABLK), 1) + j * ABLK
    valid = cols < A
    l0 = jnp.where(valid, l, 0.0)
    lm = jnp.where(valid, l, -jnp.inf)
    bm = jnp.max(lm, axis=1, keepdims=True)

    m_old = m_ref[...]
    m_new = jnp.maximum(m_old, bm)
    alpha = jnp.exp(m_old - m_new)
    e = jnp.where(valid, jnp.exp(l - m_new), 0.0)
    s_ref[...] = s_ref[...] * alpha + jnp.sum(e, axis=1, keepdims=True)
    t_ref[...] = t_ref[...] * alpha + jnp.sum(e * l0, axis=1, keepdims=True)
    m_ref[...] = m_new

    hit = cols == act_ref[...]
    la_ref[...] += jnp.sum(jnp.where(hit, l0, 0.0), axis=1, keepdims=True)

    @pl.when(j == NBLK - 1)
    def _fin():
        m = m_ref[...]
        s = s_ref[...]
        logz = m + jnp.log(s)
        lp_ref[...] = la_ref[...] - logz
        ent_ref[...] = logz - t_ref[...] / s


@jax.jit
def _ac_call(state, action, Wp, bp, Wv, bv):
    bp2 = bp.reshape(1, A)
    act2 = action.astype(jnp.int32).reshape(B, 1)
    value, lp, ent = pl.pallas_call(
        _ac_body,
        grid=(NBLK,),
        in_specs=[
            pl.BlockSpec((B, S), lambda j: (0, 0)),
            pl.BlockSpec((S, ABLK), lambda j: (0, j)),
            pl.BlockSpec((1, ABLK), lambda j: (0, j)),
            pl.BlockSpec((B, 1), lambda j: (0, 0)),
            pl.BlockSpec((S, 1), lambda j: (0, 0)),
            pl.BlockSpec((1, 1), lambda j: (0, 0)),
        ],
        out_specs=[
            pl.BlockSpec((B, 1), lambda j: (0, 0)),
            pl.BlockSpec((B, 1), lambda j: (0, 0)),
            pl.BlockSpec((B, 1), lambda j: (0, 0)),
        ],
        out_shape=[
            jax.ShapeDtypeStruct((B, 1), jnp.float32),
            jax.ShapeDtypeStruct((B, 1), jnp.float32),
            jax.ShapeDtypeStruct((B, 1), jnp.float32),
        ],
        scratch_shapes=[
            pltpu.VMEM((B, 1), jnp.float32),
            pltpu.VMEM((B, 1), jnp.float32),
            pltpu.VMEM((B, 1), jnp.float32),
            pltpu.VMEM((B, 1), jnp.float32),
        ],
    )(state, Wp, bp2, act2, Wv, bv.reshape(1, 1))
    return value, lp[:, 0], ent[:, 0]


def kernel(state, action, Wp, bp, Wv, bv):
    return _ac_call(state, action, Wp, bp, Wv, bv)


# speedup vs baseline: 1.3804x; 1.3804x over previous
import jax
import jax.numpy as jnp
from jax.experimental import pallas as pl
from jax.experimental.pallas import tpu as pltpu

B = 1024
S = 16
A = 100000
ABLK = 1024
NBLK = (A + ABLK - 1) // ABLK
NPAD = NBLK * ABLK
NSLC = ABLK // 128


def _ac_body(state_ref, wpa_ref, act_ref, wv_ref, bv_ref,
             value_ref, lp_ref, ent_ref,
             mvec_ref, lavec_ref, m_ref, s_ref, t_ref):
    p = pl.program_id(0)
    j = pl.program_id(1)

    @pl.when((p == 0) & (j == 0))
    def _init():
        mvec_ref[...] = jnp.full((B, 128), -1e30, jnp.float32)
        lavec_ref[...] = jnp.zeros((B, 128), jnp.float32)
        s_ref[...] = jnp.zeros((B, 1), jnp.float32)
        t_ref[...] = jnp.zeros((B, 1), jnp.float32)
        value_ref[...] = jnp.dot(state_ref[...], wv_ref[...],
                                 preferred_element_type=jnp.float32)

    l = jnp.dot(state_ref[...], wpa_ref[...],
                preferred_element_type=jnp.float32)

    @pl.when(p == 0)
    def _max_pass():
        cols = jax.lax.broadcasted_iota(jnp.int32, (B, ABLK), 1) + j * ABLK
        hl = jnp.where(cols == act_ref[...], l, 0.0)
        mv = mvec_ref[...]
        lav = lavec_ref[...]
        for k in range(NSLC):
            sl = l[:, k * 128:(k + 1) * 128]
            mv = jnp.maximum(mv, sl)
            lav = lav + hl[:, k * 128:(k + 1) * 128]
        mvec_ref[...] = mv
        lavec_ref[...] = lav

        @pl.when(j == NBLK - 1)
        def _finish_max():
            m_ref[...] = jnp.max(mvec_ref[...], axis=1, keepdims=True)

    @pl.when(p == 1)
    def _exp_pass():
        ones = jnp.ones((ABLK, 1), jnp.float32)
        e = jnp.exp(l - m_ref[...])
        s_ref[...] += jnp.dot(e, ones, preferred_element_type=jnp.float32)
        t_ref[...] += jnp.dot(e * l, ones,
                              preferred_element_type=jnp.float32)

        @pl.when(j == NBLK - 1)
        def _fin():
            m = m_ref[...]
            s = s_ref[...]
            logz = m + jnp.log(s)
            la = jnp.sum(lavec_ref[...], axis=1, keepdims=True)
            lp_ref[...] = la - logz
            ent_ref[...] = logz - t_ref[...] / s


@jax.jit
def _ac_call(state, action, Wp, bp, Wv, bv):
    bp_p = jnp.pad(bp, (0, NPAD - A), constant_values=-1e30)
    wpa = jnp.concatenate(
        [jnp.pad(Wp, ((0, 0), (0, NPAD - A))), bp_p[None, :]], axis=0)
    state_aug = jnp.concatenate(
        [state, jnp.ones((B, 1), jnp.float32)], axis=1)
    wv_aug = jnp.concatenate([Wv, bv[None, :]], axis=0)
    act2 = action.astype(jnp.int32).reshape(B, 1)

    value, lp, ent = pl.pallas_call(
        _ac_body,
        grid=(2, NBLK),
        in_specs=[
            pl.BlockSpec((B, S + 1), lambda p, j: (0, 0)),
            pl.BlockSpec((S + 1, ABLK), lambda p, j: (0, j)),
            pl.BlockSpec((B, 1), lambda p, j: (0, 0)),
            pl.BlockSpec((S + 1, 1), lambda p, j: (0, 0)),
            pl.BlockSpec((1, 1), lambda p, j: (0, 0)),
        ],
        out_specs=[
            pl.BlockSpec((B, 1), lambda p, j: (0, 0)),
            pl.BlockSpec((B, 1), lambda p, j: (0, 0)),
            pl.BlockSpec((B, 1), lambda p, j: (0, 0)),
        ],
        out_shape=[
            jax.ShapeDtypeStruct((B, 1), jnp.float32),
            jax.ShapeDtypeStruct((B, 1), jnp.float32),
            jax.ShapeDtypeStruct((B, 1), jnp.float32),
        ],
        scratch_shapes=[
            pltpu.VMEM((B, 128), jnp.float32),
            pltpu.VMEM((B, 128), jnp.float32),
            pltpu.VMEM((B, 1), jnp.float32),
            pltpu.VMEM((B, 1), jnp.float32),
            pltpu.VMEM((B, 1), jnp.float32),
        ],
    )(state_aug, wpa, act2, wv_aug, bv.reshape(1, 1))
    return value, lp[:, 0], ent[:, 0]


def kernel(state, action, Wp, bp, Wv, bv):
    return _ac_call(state, action, Wp, bp, Wv, bv)


# speedup vs baseline: 1.8630x; 1.3496x over previous
import jax
import jax.numpy as jnp
from jax.experimental import pallas as pl
from jax.experimental.pallas import tpu as pltpu

B = 1024
S = 16
A = 100000
ABLK = 1024
NBLK = (A + ABLK - 1) // ABLK
NPAD = NBLK * ABLK
NSLC = ABLK // 128


def _ac_body(state_ref, wpa_ref, act_ref, wv_ref, bv_ref,
             value_ref, lp_ref, ent_ref,
             mvec_ref, lavec_ref, m_ref, s_ref, t_ref):
    p = pl.program_id(0)
    j = pl.program_id(1)

    @pl.when((p == 0) & (j == 0))
    def _init():
        mvec_ref[...] = jnp.full((B, 128), -1e30, jnp.float32)
        lavec_ref[...] = jnp.zeros((B, 128), jnp.float32)
        s_ref[...] = jnp.zeros((B, 1), jnp.float32)
        t_ref[...] = jnp.zeros((B, 1), jnp.float32)
        value_ref[...] = jnp.dot(state_ref[...], wv_ref[...],
                                 preferred_element_type=jnp.float32)

    @pl.when(p == 0)
    def _max_pass():
        l = jnp.dot(state_ref[...], wpa_ref[...],
                    preferred_element_type=jnp.float32)
        cols = jax.lax.broadcasted_iota(jnp.int32, (B, ABLK), 1) + j * ABLK
        hl = jnp.where(cols == act_ref[...], l, 0.0)
        mv = mvec_ref[...]
        lav = lavec_ref[...]
        for k in range(NSLC):
            sl = l[:, k * 128:(k + 1) * 128]
            mv = jnp.maximum(mv, sl)
            lav = lav + hl[:, k * 128:(k + 1) * 128]
        mvec_ref[...] = mv
        lavec_ref[...] = lav

        @pl.when(j == NBLK - 1)
        def _finish_max():
            m_ref[...] = jnp.max(mvec_ref[...], axis=1, keepdims=True)

    @pl.when(p == 1)
    def _exp_pass():
        lhs = jnp.concatenate(
            [state_ref[:, 0:S + 1], -m_ref[...]], axis=1)
        l2 = jnp.dot(lhs, wpa_ref[...],
                     preferred_element_type=jnp.float32)
        ones = jnp.ones((ABLK, 1), jnp.float32)
        e = jnp.exp(l2)
        s_ref[...] += jnp.dot(e, ones, preferred_element_type=jnp.float32)
        t_ref[...] += jnp.dot(e * l2, ones,
                              preferred_element_type=jnp.float32)

        @pl.when(j == NBLK - 1)
        def _fin():
            s = s_ref[...]
            logs = jnp.log(s)
            la = jnp.sum(lavec_ref[...], axis=1, keepdims=True)
            lp_ref[...] = la - m_ref[...] - logs
            ent_ref[...] = logs - t_ref[...] / s


@jax.jit
def _ac_call(state, action, Wp, bp, Wv, bv):
    bp_p = jnp.pad(bp, (0, NPAD - A), constant_values=-1e30)
    wpa = jnp.concatenate(
        [jnp.pad(Wp, ((0, 0), (0, NPAD - A))),
         bp_p[None, :],
         jnp.ones((1, NPAD), jnp.float32)], axis=0)
    state_aug = jnp.concatenate(
        [state, jnp.ones((B, 1), jnp.float32),
         jnp.zeros((B, 1), jnp.float32)], axis=1)
    wv_aug = jnp.concatenate(
        [Wv, bv[None, :], jnp.zeros((1, 1), jnp.float32)], axis=0)
    act2 = action.astype(jnp.int32).reshape(B, 1)

    value, lp, ent = pl.pallas_call(
        _ac_body,
        grid=(2, NBLK),
        in_specs=[
            pl.BlockSpec((B, S + 2), lambda p, j: (0, 0)),
            pl.BlockSpec((S + 2, ABLK), lambda p, j: (0, j)),
            pl.BlockSpec((B, 1), lambda p, j: (0, 0)),
            pl.BlockSpec((S + 2, 1), lambda p, j: (0, 0)),
            pl.BlockSpec((1, 1), lambda p, j: (0, 0)),
        ],
        out_specs=[
            pl.BlockSpec((B, 1), lambda p, j: (0, 0)),
            pl.BlockSpec((B, 1), lambda p, j: (0, 0)),
            pl.BlockSpec((B, 1), lambda p, j: (0, 0)),
        ],
        out_shape=[
            jax.ShapeDtypeStruct((B, 1), jnp.float32),
            jax.ShapeDtypeStruct((B, 1), jnp.float32),
            jax.ShapeDtypeStruct((B, 1), jnp.float32),
        ],
        scratch_shapes=[
            pltpu.VMEM((B, 128), jnp.float32),
            pltpu.VMEM((B, 128), jnp.float32),
            pltpu.VMEM((B, 1), jnp.float32),
            pltpu.VMEM((B, 1), jnp.float32),
            pltpu.VMEM((B, 1), jnp.float32),
        ],
    )(state_aug, wpa, act2, wv_aug, bv.reshape(1, 1))
    return value, lp[:, 0], ent[:, 0]


def kernel(state, action, Wp, bp, Wv, bv):
    return _ac_call(state, action, Wp, bp, Wv, bv)
